# baseline (speedup 1.0000x reference)
import functools

import jax
import jax.numpy as jnp
from jax import lax
from jax.experimental import pallas as pl
from jax.experimental.pallas import tpu as pltpu
from jax.experimental.pallas import tpu_sc as plsc

B = 4096
N_DEEP = 26
DEEP_VOCAB = 100000
DEEP_DIM = 100
ITEM_VOCAB = 1000000
ITEM_DIM = 64
NUM_WIDE = 26
CROSS = 128

NUM_CORES = 2
NUM_SUBCORES = 16
NW = NUM_CORES * NUM_SUBCORES
BPW = B // NW

VC = 12800


def _pack_rows(r):
    n = r.shape[0]
    r2 = r.reshape(n // 2, 2, CROSS)
    a = lax.bitcast_convert_type(r2[:, 0, :].astype(jnp.bfloat16), jnp.uint16)
    bm = lax.bitcast_convert_type(r2[:, 1, :].astype(jnp.bfloat16), jnp.uint16)
    word = a.astype(jnp.uint32) | (bm.astype(jnp.uint32) << 16)
    return lax.bitcast_convert_type(word, jnp.float32)


def _project_deep_body(t_ref, w_ref, p_ref):
    tb = t_ref[0].astype(jnp.bfloat16)
    w = w_ref[0].astype(jnp.bfloat16)
    r = lax.dot_general(
        tb, w, (((0,), (0,)), ((), ())),
        preferred_element_type=jnp.float32,
    )
    p_ref[0] = _pack_rows(r)


def _project_deep(deep_t, wd):
    nvb = (DEEP_VOCAB + VC - 1) // VC
    return pl.pallas_call(
        _project_deep_body,
        grid=(N_DEEP, nvb),
        in_specs=[
            pl.BlockSpec((1, DEEP_DIM, VC), lambda i, v: (i, 0, v)),
            pl.BlockSpec((1, DEEP_DIM, CROSS), lambda i, v: (i, 0, 0)),
        ],
        out_specs=pl.BlockSpec((1, VC // 2, CROSS), lambda i, v: (i, v, 0)),
        out_shape=jax.ShapeDtypeStruct(
            (N_DEEP, DEEP_VOCAB // 2, CROSS), jnp.float32
        ),
        compiler_params=pltpu.CompilerParams(
            dimension_semantics=("parallel", "parallel"),
        ),
    )(deep_t, wd)


def _project_item_body(t_ref, w_ref, p_ref):
    tb = t_ref[...].astype(jnp.bfloat16)
    w = w_ref[...].astype(jnp.bfloat16)
    r = lax.dot_general(
        tb, w, (((0,), (0,)), ((), ())),
        preferred_element_type=jnp.float32,
    )
    p_ref[...] = _pack_rows(r)


def _project_item(item_t, wdev):
    nvb = (ITEM_VOCAB + VC - 1) // VC
    return pl.pallas_call(
        _project_item_body,
        grid=(nvb,),
        in_specs=[
            pl.BlockSpec((ITEM_DIM, VC), lambda v: (0, v)),
            pl.BlockSpec((ITEM_DIM, CROSS), lambda v: (0, 0)),
        ],
        out_specs=pl.BlockSpec((VC // 2, CROSS), lambda v: (v, 0)),
        out_shape=jax.ShapeDtypeStruct(
            (ITEM_VOCAB // 2, CROSS), jnp.float32
        ),
        compiler_params=pltpu.CompilerParams(
            dimension_semantics=("parallel",),
        ),
    )(item_t, wdev)


@functools.lru_cache(maxsize=1)
def _sc_gather_build():
    mesh = plsc.VectorSubcoreMesh(core_axis_name="c", subcore_axis_name="s")

    @functools.partial(
        pl.kernel,
        mesh=mesh,
        out_type=(
            jax.ShapeDtypeStruct((N_DEEP, B, CROSS), jnp.float32),
            jax.ShapeDtypeStruct((B, CROSS), jnp.float32),
        ),
        scratch_types=[
            pltpu.VMEM((BPW,), jnp.int32),
            pltpu.VMEM((BPW,), jnp.int32),
            pltpu.VMEM((BPW, CROSS), jnp.float32),
            pltpu.VMEM((BPW, CROSS), jnp.float32),
            pltpu.SemaphoreType.DMA,
        ],
    )
    def sc_gather(
        p_deep_hbm,
        deep_idx_hbm,
        p_item_hbm,
        dev_idx_hbm,
        deep_out_hbm,
        dev_out_hbm,
        dev_idx_v,
        cur_idx_v,
        rows_v,
        item_rows_v,
        gsem,
    ):
        wid = lax.axis_index("s") * NUM_CORES + lax.axis_index("c")
        base = wid * BPW

        pltpu.sync_copy(dev_idx_hbm.at[pl.ds(base, BPW)], dev_idx_v)
        pltpu.async_copy(p_item_hbm.at[dev_idx_v], item_rows_v, gsem).wait()
        pltpu.sync_copy(item_rows_v, dev_out_hbm.at[pl.ds(base, BPW)])

        def body(t, _):
            pltpu.sync_copy(deep_idx_hbm.at[pl.ds(t * B + base, BPW)], cur_idx_v)
            pltpu.async_copy(p_deep_hbm.at[cur_idx_v], rows_v, gsem).wait()
            pltpu.sync_copy(rows_v, deep_out_hbm.at[t, pl.ds(base, BPW)])
            return _

        lax.fori_loop(0, N_DEEP, body, None)

    return sc_gather


def _unpack_sel(packed, par):
    w = lax.bitcast_convert_type(packed, jnp.uint32)
    lo = lax.bitcast_convert_type(w.astype(jnp.uint16), jnp.bfloat16)
    hi = lax.bitcast_convert_type((w >> 16).astype(jnp.uint16), jnp.bfloat16)
    return jnp.where(par > 0.5, hi.astype(jnp.float32), lo.astype(jnp.float32))


def _combine_body(g_ref, par_ref, dev_ref, dpar_ref, wide_ref, wwide_ref,
                  b_ref, out_ref):
    g = g_ref[...]
    p = par_ref[...]
    acc = jnp.sum(_unpack_sel(g, p), axis=0)
    acc = acc + _unpack_sel(dev_ref[...], dpar_ref[...])

    wblk = wide_ref[...]
    mean = jnp.mean(wblk, axis=0, keepdims=True)
    var = jnp.mean(jnp.square(wblk - mean), axis=0, keepdims=True)
    wn = (wblk - mean) * lax.rsqrt(var + 1e-5)
    wide_part = lax.dot_general(
        wn, wwide_ref[...], (((0,), (0,)), ((), ())),
        preferred_element_type=jnp.float32,
        precision=lax.Precision.HIGHEST,
    )
    out_ref[...] = acc + wide_part + b_ref[...]


def _combine(gathered, par_t, dev, dev_par, wide_in, wwide, b2):
    BT = 512
    return pl.pallas_call(
        _combine_body,
        grid=(B // BT,),
        in_specs=[
            pl.BlockSpec((N_DEEP, BT, CROSS), lambda bb: (0, bb, 0)),
            pl.BlockSpec((N_DEEP, BT, 1), lambda bb: (0, bb, 0)),
            pl.BlockSpec((BT, CROSS), lambda bb: (bb, 0)),
            pl.BlockSpec((BT, 1), lambda bb: (bb, 0)),
            pl.BlockSpec((NUM_WIDE, BT), lambda bb: (0, bb)),
            pl.BlockSpec((NUM_WIDE, CROSS), lambda bb: (0, 0)),
            pl.BlockSpec((1, CROSS), lambda bb: (0, 0)),
        ],
        out_specs=pl.BlockSpec((BT, CROSS), lambda bb: (bb, 0)),
        out_shape=jax.ShapeDtypeStruct((B, CROSS), jnp.float32),
        compiler_params=pltpu.CompilerParams(
            dimension_semantics=("parallel",),
        ),
    )(gathered, par_t, dev, dev_par, wide_in, wwide, b2)


def kernel(deep_in, wide_in, device_in, deep_tables, item_table, ln_gamma, ln_beta, W, b):
    deep_in = deep_in.astype(jnp.int32)
    device_in = device_in.astype(jnp.int32)
    deep_t = jnp.transpose(deep_tables, (0, 2, 1))
    item_t = jnp.transpose(item_table)

    wd = W[: N_DEEP * DEEP_DIM].reshape(N_DEEP, DEEP_DIM, CROSS)
    wdev = W[N_DEEP * DEEP_DIM : N_DEEP * DEEP_DIM + ITEM_DIM]
    w_wide_raw = W[N_DEEP * DEEP_DIM + ITEM_DIM :]
    wwide = ln_gamma[:, None] * w_wide_raw
    b2 = (b + ln_beta @ w_wide_raw).reshape(1, CROSS)

    p_deep = _project_deep(deep_t, wd).reshape(N_DEEP * DEEP_VOCAB // 2, CROSS)
    p_item = _project_item(item_t, wdev)

    offs = (jnp.arange(N_DEEP, dtype=jnp.int32) * DEEP_VOCAB)[:, None]
    flat_idx = (deep_in + offs).reshape(N_DEEP * B)
    half_idx = flat_idx >> 1
    par_t = (deep_in & 1).astype(jnp.float32)[:, :, None]
    dev_par = (device_in & 1).astype(jnp.float32)[:, None]
    gathered, dev = _sc_gather_build()(
        p_deep, half_idx, p_item, device_in >> 1
    )

    return _combine(gathered, par_t, dev, dev_par, wide_in, wwide, b2)


# speedup vs baseline: 2.7000x; 2.7000x over previous
import functools

import jax
import jax.numpy as jnp
from jax import lax
from jax.experimental import pallas as pl
from jax.experimental.pallas import tpu as pltpu
from jax.experimental.pallas import tpu_sc as plsc

B = 4096
N_DEEP = 26
DEEP_VOCAB = 100000
DEEP_DIM = 100
ITEM_VOCAB = 1000000
ITEM_DIM = 64
NUM_WIDE = 26
CROSS = 128

NUM_CORES = 2
NUM_SUBCORES = 16
NW = NUM_CORES * NUM_SUBCORES
BPW = B // NW

VC = 12800


HALF_DEEP = N_DEEP // 2


def _pack_two(r1, r2):
    a = lax.bitcast_convert_type(r1.astype(jnp.bfloat16), jnp.uint16)
    bm = lax.bitcast_convert_type(r2.astype(jnp.bfloat16), jnp.uint16)
    word = a.astype(jnp.uint32) | (bm.astype(jnp.uint32) << 16)
    return lax.bitcast_convert_type(word, jnp.float32)


def _project_deep_body(t1_ref, t2_ref, w1_ref, w2_ref, p_ref):
    dims = (((0,), (0,)), ((), ()))
    r1 = lax.dot_general(
        t1_ref[0].astype(jnp.bfloat16), w1_ref[0].astype(jnp.bfloat16),
        dims, preferred_element_type=jnp.float32,
    )
    r2 = lax.dot_general(
        t2_ref[0].astype(jnp.bfloat16), w2_ref[0].astype(jnp.bfloat16),
        dims, preferred_element_type=jnp.float32,
    )
    p_ref[0] = _pack_two(r1, r2)


def _project_deep(deep_t, wd):
    nvb = (DEEP_VOCAB + VC - 1) // VC
    return pl.pallas_call(
        _project_deep_body,
        grid=(HALF_DEEP, nvb),
        in_specs=[
            pl.BlockSpec((1, DEEP_DIM, VC), lambda i, v: (i, 0, v)),
            pl.BlockSpec((1, DEEP_DIM, VC), lambda i, v: (i + HALF_DEEP, 0, v)),
            pl.BlockSpec((1, DEEP_DIM, CROSS), lambda i, v: (i, 0, 0)),
            pl.BlockSpec((1, DEEP_DIM, CROSS), lambda i, v: (i + HALF_DEEP, 0, 0)),
        ],
        out_specs=pl.BlockSpec((1, VC, CROSS), lambda i, v: (i, v, 0)),
        out_shape=jax.ShapeDtypeStruct(
            (HALF_DEEP, DEEP_VOCAB, CROSS), jnp.float32
        ),
        compiler_params=pltpu.CompilerParams(
            dimension_semantics=("parallel", "parallel"),
        ),
    )(deep_t, deep_t, wd, wd)


def _project_item_body(t_ref, w_ref, p_ref):
    tb = t_ref[...].astype(jnp.bfloat16)
    w = w_ref[...].astype(jnp.bfloat16)
    p_ref[...] = lax.dot_general(
        tb, w, (((0,), (0,)), ((), ())),
        preferred_element_type=jnp.float32,
    )


def _project_item(item_t, wdev):
    nvb = (ITEM_VOCAB + VC - 1) // VC
    return pl.pallas_call(
        _project_item_body,
        grid=(nvb,),
        in_specs=[
            pl.BlockSpec((ITEM_DIM, VC), lambda v: (0, v)),
            pl.BlockSpec((ITEM_DIM, CROSS), lambda v: (0, 0)),
        ],
        out_specs=pl.BlockSpec((VC, CROSS), lambda v: (v, 0)),
        out_shape=jax.ShapeDtypeStruct(
            (ITEM_VOCAB, CROSS), jnp.float32
        ),
        compiler_params=pltpu.CompilerParams(
            dimension_semantics=("parallel",),
        ),
    )(item_t, wdev)


@functools.lru_cache(maxsize=1)
def _sc_gather_build():
    mesh = plsc.VectorSubcoreMesh(core_axis_name="c", subcore_axis_name="s")

    @functools.partial(
        pl.kernel,
        mesh=mesh,
        out_type=(
            jax.ShapeDtypeStruct((N_DEEP, B, CROSS), jnp.float32),
            jax.ShapeDtypeStruct((B, CROSS), jnp.float32),
        ),
        scratch_types=[
            pltpu.VMEM((BPW,), jnp.int32),
            pltpu.VMEM((BPW,), jnp.int32),
            pltpu.VMEM((BPW, CROSS), jnp.float32),
            pltpu.VMEM((BPW, CROSS), jnp.float32),
            pltpu.SemaphoreType.DMA,
        ],
    )
    def sc_gather(
        p_deep_hbm,
        deep_idx_hbm,
        p_item_hbm,
        dev_idx_hbm,
        deep_out_hbm,
        dev_out_hbm,
        dev_idx_v,
        cur_idx_v,
        rows_v,
        item_rows_v,
        gsem,
    ):
        wid = lax.axis_index("s") * NUM_CORES + lax.axis_index("c")
        base = wid * BPW

        pltpu.sync_copy(dev_idx_hbm.at[pl.ds(base, BPW)], dev_idx_v)
        pltpu.async_copy(p_item_hbm.at[dev_idx_v], item_rows_v, gsem).wait()
        pltpu.sync_copy(item_rows_v, dev_out_hbm.at[pl.ds(base, BPW)])

        def body(t, _):
            pltpu.sync_copy(deep_idx_hbm.at[pl.ds(t * B + base, BPW)], cur_idx_v)
            pltpu.async_copy(p_deep_hbm.at[cur_idx_v], rows_v, gsem).wait()
            pltpu.sync_copy(rows_v, deep_out_hbm.at[t, pl.ds(base, BPW)])
            return _

        lax.fori_loop(0, N_DEEP, body, None)

    return sc_gather


def _combine_body(g_ref, dev_ref, wide_ref, wwide_ref, b_ref, out_ref):
    g = g_ref[...]
    w = lax.bitcast_convert_type(g, jnp.uint32)
    lo = lax.bitcast_convert_type(
        w[:HALF_DEEP].astype(jnp.uint16), jnp.bfloat16).astype(jnp.float32)
    hi = lax.bitcast_convert_type(
        (w[HALF_DEEP:] >> 16).astype(jnp.uint16), jnp.bfloat16).astype(jnp.float32)
    acc = jnp.sum(lo, axis=0) + jnp.sum(hi, axis=0)
    acc = acc + dev_ref[...]

    wblk = wide_ref[...]
    mean = jnp.mean(wblk, axis=0, keepdims=True)
    var = jnp.mean(jnp.square(wblk - mean), axis=0, keepdims=True)
    wn = (wblk - mean) * lax.rsqrt(var + 1e-5)
    wide_part = lax.dot_general(
        wn, wwide_ref[...], (((0,), (0,)), ((), ())),
        preferred_element_type=jnp.float32,
        precision=lax.Precision.HIGHEST,
    )
    out_ref[...] = acc + wide_part + b_ref[...]


def _combine(gathered, dev, wide_in, wwide, b2):
    BT = 512
    return pl.pallas_call(
        _combine_body,
        grid=(B // BT,),
        in_specs=[
            pl.BlockSpec((N_DEEP, BT, CROSS), lambda bb: (0, bb, 0)),
            pl.BlockSpec((BT, CROSS), lambda bb: (bb, 0)),
            pl.BlockSpec((NUM_WIDE, BT), lambda bb: (0, bb)),
            pl.BlockSpec((NUM_WIDE, CROSS), lambda bb: (0, 0)),
            pl.BlockSpec((1, CROSS), lambda bb: (0, 0)),
        ],
        out_specs=pl.BlockSpec((BT, CROSS), lambda bb: (bb, 0)),
        out_shape=jax.ShapeDtypeStruct((B, CROSS), jnp.float32),
        compiler_params=pltpu.CompilerParams(
            dimension_semantics=("parallel",),
        ),
    )(gathered, dev, wide_in, wwide, b2)


def kernel(deep_in, wide_in, device_in, deep_tables, item_table, ln_gamma, ln_beta, W, b):
    deep_in = deep_in.astype(jnp.int32)
    device_in = device_in.astype(jnp.int32)
    deep_t = jnp.transpose(deep_tables, (0, 2, 1))
    item_t = jnp.transpose(item_table)

    wd = W[: N_DEEP * DEEP_DIM].reshape(N_DEEP, DEEP_DIM, CROSS)
    wdev = W[N_DEEP * DEEP_DIM : N_DEEP * DEEP_DIM + ITEM_DIM]
    w_wide_raw = W[N_DEEP * DEEP_DIM + ITEM_DIM :]
    wwide = ln_gamma[:, None] * w_wide_raw
    b2 = (b + ln_beta @ w_wide_raw).reshape(1, CROSS)

    p_deep = _project_deep(deep_t, wd).reshape(HALF_DEEP * DEEP_VOCAB, CROSS)
    p_item = _project_item(item_t, wdev)

    offs = ((jnp.arange(N_DEEP, dtype=jnp.int32) % HALF_DEEP) * DEEP_VOCAB)[:, None]
    flat_idx = (deep_in + offs).reshape(N_DEEP * B)
    gathered, dev = _sc_gather_build()(
        p_deep, flat_idx, p_item, device_in
    )

    return _combine(gathered, dev, wide_in, wwide, b2)
